# baseline (speedup 1.0000x reference)
import functools

import jax
import jax.numpy as jnp
from jax import lax
from jax.experimental import pallas as pl
from jax.experimental.pallas import tpu as pltpu
from jax.experimental.pallas import tpu_sc as plsc

N, H, W, K, F, C = 4, 512, 512, 8, 100000, 16
B = N * H * W
NW = 32
P = 1024
NCH = B // (NW * P)
CPN = (H // 2) * (W // 512)
GSZ = 128
NG = P // GSZ
L = 16


def _tex_kernel(pix_hbm, bary_hbm, table_hbm, out_hbm,
                pix_v, bary_v, idx_v, rows_v, out_v, semb, semg):
    wid = lax.axis_index("s") * 2 + lax.axis_index("c")
    iota = lax.iota(jnp.int32, L)
    zero = jnp.zeros((L,), jnp.int32)

    def chunk_body(ci, carry):
        gc = wid * NCH + ci
        n = gc // CPN
        h0 = pl.multiple_of((gc % CPN) * 2, 2)

        bary_cp = pltpu.async_copy(
            bary_hbm.at[n, pl.ds(h0, 2), pl.ds(0, W), pl.ds(0, 1), pl.ds(0, 3)],
            bary_v, semb)
        pltpu.sync_copy(
            pix_hbm.at[n, pl.ds(h0, 2), pl.ds(0, W), pl.ds(0, 1)], pix_v)

        gather_cps = []
        for j in range(NG):
            def ex_body(g, _, j=j):
                pv = jnp.full((L,), j * GSZ + g * L, jnp.int32) + iota
                vals = plsc.load_gather(
                    pix_v, [pv >> 9, pv & (W - 1), zero])
                idx_v[j, pl.ds(g * L, L)] = vals
                return 0
            lax.fori_loop(0, GSZ // L, ex_body, 0)
            gather_cps.append(
                pltpu.async_copy(table_hbm.at[idx_v.at[j]], rows_v.at[j], semg))

        bary_cp.wait()
        for cp in gather_cps:
            cp.wait()

        for j in range(NG):
            def px_body(g, _, j=j):
                pv = jnp.full((L,), j * GSZ + g * L, jnp.int32) + iota
                pin = jnp.full((L,), g * L, jnp.int32) + iota
                rv = pv >> 9
                wv = pv & (W - 1)
                jc = jnp.full((L,), j, jnp.int32)
                b0 = plsc.load_gather(bary_v, [rv, wv, zero, zero])
                b1 = plsc.load_gather(bary_v, [rv, wv, zero, zero + 1])
                b2 = plsc.load_gather(bary_v, [rv, wv, zero, zero + 2])
                for c in range(C):
                    r0 = plsc.load_gather(rows_v, [jc, pin, zero, zero + c])
                    r1 = plsc.load_gather(rows_v, [jc, pin, zero + 1, zero + c])
                    r2 = plsc.load_gather(rows_v, [jc, pin, zero + 2, zero + c])
                    acc = b0 * r0 + b1 * r1 + b2 * r2
                    plsc.store_scatter(out_v, [rv, wv, zero + c], acc)
                return 0
            lax.fori_loop(0, GSZ // L, px_body, 0)

        pltpu.sync_copy(out_v, out_hbm.at[n, pl.ds(h0, 2)])
        return carry

    lax.fori_loop(0, NCH, chunk_body, 0)


@jax.jit
def _run(pix, bary, table):
    mesh = plsc.VectorSubcoreMesh(core_axis_name="c", subcore_axis_name="s")
    f = functools.partial(
        pl.kernel,
        mesh=mesh,
        compiler_params=pltpu.CompilerParams(
            needs_layout_passes=False, use_tc_tiling_on_sc=False),
        out_type=jax.ShapeDtypeStruct((N, H, W, C), jnp.float32),
        scratch_types=[
            pltpu.VMEM((2, W, 1), jnp.int32),
            pltpu.VMEM((2, W, 1, 3), jnp.float32),
            pltpu.VMEM((NG, GSZ), jnp.int32),
            pltpu.VMEM((NG, GSZ, 3, C), jnp.float32),
            pltpu.VMEM((2, W, C), jnp.float32),
            pltpu.SemaphoreType.DMA,
            pltpu.SemaphoreType.DMA,
        ],
    )(_tex_kernel)
    return f(pix, bary, table)


def kernel(bary_coords, pix_to_face, face_textures):
    return _run(pix_to_face.astype(jnp.int32), bary_coords, face_textures)


# speedup vs baseline: 30.4228x; 30.4228x over previous
import functools

import jax
import jax.numpy as jnp
from jax import lax
from jax.experimental import pallas as pl
from jax.experimental.pallas import tpu as pltpu
from jax.experimental.pallas import tpu_sc as plsc

N, H, W, K, F, C = 4, 512, 512, 8, 100000, 16
B = N * H * W
NW = 32
PXW = B // NW
P = 1024
NCH = PXW // P
GSZ = 128
NU = 4
L = 16


def _tex_kernel(pix_hbm, bary_hbm, table_hbm, out_hbm,
                pixi_v, baryi_v, pix_rows, bary_rows, rows_a, rows_b, out_v,
                semp, semb, semg, semo):
    wid = lax.axis_index("s") * 2 + lax.axis_index("c")
    iota = lax.iota(jnp.int32, L)
    cpat = jnp.where(iota < 12, (iota >> 2) * 32 + (iota & 3) * 8, 0)
    pix_pat = jnp.where(iota < 8, iota * 8, 0)

    def fire_io(ci):
        gc = wid * NCH + ci
        nh = (gc >> 8) * 512 + (gc & 255) * 2
        q = ci & 1
        pixi_v[q] = jnp.full((L,), nh * 32, jnp.int32) + pix_pat
        baryi_v[q, pl.ds(0, L)] = jnp.full((L,), nh * 96, jnp.int32) + cpat
        baryi_v[q, pl.ds(L, L)] = (
            jnp.full((L,), (nh + 1) * 96, jnp.int32) + cpat)
        pltpu.async_copy(pix_hbm.at[pixi_v.at[q]], pix_rows.at[q], semp)
        pltpu.async_copy(bary_hbm.at[baryi_v.at[q]], bary_rows.at[q], semb)

    fire_io(0)

    def chunk_body(ci, carry):
        gc = wid * NCH + ci
        n = gc >> 8
        h0 = (gc & 255) * 2
        nh = n * 512 + h0
        par = ci & 1
        orow = pl.multiple_of(nh * 16, 32)

        pltpu.make_async_copy(
            pix_hbm.at[pixi_v.at[par]], pix_rows.at[par], semp).wait()

        @pl.when(ci >= 2)
        def _():
            pltpu.make_async_copy(
                out_v.at[par], out_hbm.at[pl.ds(orow, 32)], semo).wait()

        bufs = [rows_a, rows_b]
        cps = [
            pltpu.async_copy(
                table_hbm.at[pix_rows.at[par, b]], bufs[0].at[b], semg)
            for b in range(2)
        ]

        @pl.when(ci + 1 < NCH)
        def _():
            fire_io(ci + 1)
        for u in range(NU):
            rbuf = bufs[u % 2]
            if u < NU - 1:
                nxt = [
                    pltpu.async_copy(
                        table_hbm.at[pix_rows.at[par, 2 * (u + 1) + b]],
                        bufs[(u + 1) % 2].at[b], semg)
                    for b in range(2)
                ]
            for cp in cps:
                cp.wait()
            if u == 0:
                pltpu.make_async_copy(
                    bary_hbm.at[baryi_v.at[par]], bary_rows.at[par],
                    semb).wait()
            parv = jnp.full((L,), par, jnp.int32)
            for b in range(2):
                blk = 2 * u + b
                hloc = blk // 4
                wt = blk % 4
                crows = jnp.full((L,), hloc * 16, jnp.int32) + iota

                @plsc.parallel_loop(0, GSZ, unroll=8)
                def px_body(i, b=b, blk=blk, hloc=hloc, wt=wt, crows=crows,
                            rbuf=rbuf):
                    iv = jnp.full((L,), i, jnp.int32)
                    b0 = plsc.load_gather(
                        bary_rows,
                        [parv, jnp.full((L,), hloc * 16 + wt, jnp.int32), iv])
                    b1 = plsc.load_gather(
                        bary_rows,
                        [parv, jnp.full((L,), hloc * 16 + 4 + wt, jnp.int32), iv])
                    b2 = plsc.load_gather(
                        bary_rows,
                        [parv, jnp.full((L,), hloc * 16 + 8 + wt, jnp.int32), iv])
                    r0 = rbuf[b, i, pl.ds(0, L)]
                    r1 = rbuf[b, i, pl.ds(16, L)]
                    r2 = rbuf[b, i, pl.ds(32, L)]
                    plsc.store_scatter(
                        out_v,
                        [parv, crows, jnp.full((L,), wt * 128, jnp.int32) + iv],
                        b0 * r0 + b1 * r1 + b2 * r2)
            if u < NU - 1:
                cps = nxt

        pltpu.async_copy(out_v.at[par], out_hbm.at[pl.ds(orow, 32)], semo)
        return carry

    lax.fori_loop(0, NCH, chunk_body, 0)
    for _ in range(2):
        pltpu.make_async_copy(
            out_v.at[0], out_hbm.at[pl.ds(0, 32)], semo).wait()


@jax.jit
def _run(pix, bary, table):
    mesh = plsc.VectorSubcoreMesh(core_axis_name="c", subcore_axis_name="s")
    f = functools.partial(
        pl.kernel,
        mesh=mesh,
        compiler_params=pltpu.CompilerParams(
            needs_layout_passes=False, use_tc_tiling_on_sc=True),
        out_type=jax.ShapeDtypeStruct((N * H * C, W), jnp.float32),
        scratch_types=[
            pltpu.VMEM((2, L), jnp.int32),
            pltpu.VMEM((2, 2 * L), jnp.int32),
            pltpu.VMEM((2, L, 128), jnp.int32),
            pltpu.VMEM((2, 2 * L, 128), jnp.float32),
            pltpu.VMEM((2, GSZ, 128), jnp.float32),
            pltpu.VMEM((2, GSZ, 128), jnp.float32),
            pltpu.VMEM((2, 2 * C, W), jnp.float32),
            pltpu.SemaphoreType.DMA,
            pltpu.SemaphoreType.DMA,
            pltpu.SemaphoreType.DMA,
            pltpu.SemaphoreType.DMA,
        ],
    )(_tex_kernel)
    return f(pix, bary, table)


def kernel(bary_coords, pix_to_face, face_textures):
    pix = (pix_to_face.astype(jnp.int32)
           .reshape(N, H, 4, 128, K)
           .transpose(0, 1, 2, 4, 3)
           .reshape(B * K // 128, 128))
    bary = (bary_coords
            .reshape(N, H, 4, 128, K, 3)
            .transpose(0, 1, 5, 2, 4, 3)
            .reshape(B * K * 3 // 128, 128))
    table = jnp.pad(face_textures.reshape(F, 3 * C), ((0, 0), (0, 128 - 3 * C)))
    out = _run(pix, bary, table)
    return jnp.transpose(out.reshape(N, H, C, W), (0, 1, 3, 2))
